# baseline (speedup 1.0000x reference)
import functools

import jax
import jax.numpy as jnp
from jax import lax
from jax.experimental import pallas as pl
from jax.experimental.pallas import tpu as pltpu
from jax.experimental.pallas import tpu_sc as plsc

_NC = 2
_NS = 16
_NW = _NC * _NS
_L = 16

_MESH = dict(core_axis_name="c", subcore_axis_name="s",
             num_cores=_NC, num_subcores=_NS)

_TC_TILED = pltpu.CompilerParams(use_tc_tiling_on_sc=True,
                                 needs_layout_passes=False)

_K0_CHUNK = 128


@functools.cache
def _make_untile(V, D):
    n_full = V // _K0_CHUNK
    n_pairs_total = n_full // 2
    pair_lo = n_pairs_total // _NW
    n_extra = n_pairs_total % _NW
    odd_chunk = n_full % 2
    tail = V % _K0_CHUNK
    words = _K0_CHUNK * D

    def body(wT_hbm, tail_hbm, out_hbm, buf0, buf1, st0, st1, sem):
        wid = lax.axis_index("s") * _NC + lax.axis_index("c")
        iota = lax.iota(jnp.int32, _L)

        extra = jnp.where(wid < n_extra, 1, 0)
        n_pairs = pair_lo + extra
        base = wid * pair_lo + jnp.minimum(wid, n_extra)

        def fire(c, buf):
            pltpu.async_copy(
                wT_hbm.at[:, pl.ds(c * _K0_CHUNK, _K0_CHUNK)], buf, sem)

        def drain(buf):
            pltpu.make_async_copy(
                wT_hbm.at[:, pl.ds(0, _K0_CHUNK)], buf, sem).wait()

        def emit(c, buf, st):
            def inner(pb, carry):
                for j in range(_L):
                    col = pb * _L + j
                    for h in range(D // _L):
                        vals = plsc.load_gather(
                            buf,
                            [iota + h * _L, jnp.full((_L,), col, jnp.int32)])
                        st[pl.ds(col * D + h * _L, _L)] = vals
                return carry
            lax.fori_loop(0, _K0_CHUNK // _L, inner, 0)
            pltpu.sync_copy(st, out_hbm.at[pl.ds(c * _K0_CHUNK * D, words)])

        fire(2 * base, buf0)

        def step(j, carry):
            c = 2 * (base + j)
            fire(c + 1, buf1)
            drain(buf0)
            emit(c, buf0, st0)
            nxt = 2 * (base + jnp.minimum(j + 1, n_pairs - 1))
            fire(nxt, buf0)
            drain(buf1)
            emit(c + 1, buf1, st1)
            return carry

        lax.fori_loop(0, n_pairs, step, 0)
        drain(buf0)

        if odd_chunk:
            @pl.when(wid == 1)
            def _():
                c = n_full - 1
                fire(c, buf0)
                drain(buf0)
                emit(c, buf0, st0)

        if tail:
            @pl.when(wid == 0)
            def _():
                pltpu.sync_copy(tail_hbm, st0.at[pl.ds(0, tail * D)])
                pltpu.sync_copy(st0.at[pl.ds(0, tail * D)],
                                out_hbm.at[pl.ds(n_full * _K0_CHUNK * D,
                                                 tail * D)])

    return pl.kernel(
        body,
        out_type=jax.ShapeDtypeStruct((V * D,), jnp.float32),
        mesh=plsc.VectorSubcoreMesh(**_MESH),
        compiler_params=_TC_TILED,
        scratch_types=[
            pltpu.VMEM((D, _K0_CHUNK), jnp.float32),
            pltpu.VMEM((D, _K0_CHUNK), jnp.float32),
            pltpu.VMEM((words,), jnp.float32),
            pltpu.VMEM((words,), jnp.float32),
            pltpu.SemaphoreType.DMA,
        ],
    )


_CHUNK = 1024
_GATHER = 128


@functools.cache
def _make_gather(N, V, D):
    per_w = N // _NW
    n_chunks = per_w // _CHUNK
    k = _CHUNK // _GATHER

    def body(idx_hbm, w_hbm, out_hbm, idx_v, rows_v, sem):
        wid = lax.axis_index("s") * _NC + lax.axis_index("c")
        base = wid * per_w

        def step(g, carry):
            off = base + g * _CHUNK
            pltpu.sync_copy(idx_hbm.at[pl.ds(off, _CHUNK)], idx_v)
            descs = [
                pltpu.async_copy(
                    w_hbm.at[idx_v.at[pl.ds(j * _GATHER, _GATHER)]],
                    rows_v.at[pl.ds(j * _GATHER, _GATHER)],
                    sem)
                for j in range(k)
            ]
            for d in descs:
                d.wait()
            pltpu.sync_copy(rows_v, out_hbm.at[pl.ds(off, _CHUNK)])
            return carry

        lax.fori_loop(0, n_chunks, step, 0)

    return pl.kernel(
        body,
        out_type=jax.ShapeDtypeStruct((N, D), jnp.float32),
        mesh=plsc.VectorSubcoreMesh(**_MESH),
        compiler_params=pltpu.CompilerParams(use_tc_tiling_on_sc=False),
        scratch_types=[
            pltpu.VMEM((_CHUNK,), jnp.int32),
            pltpu.VMEM((_CHUNK, D), jnp.float32),
            pltpu.SemaphoreType.DMA,
        ],
    )


_BLK = 128


@functools.cache
def _make_retile(F, B, D):
    n_pairs = F * (B // _BLK) // _NW
    blocks_per_f = B // _BLK
    words = _BLK * D

    def body(g_hbm, out_hbm, buf0, buf1, st, sem):
        wid = lax.axis_index("s") * _NC + lax.axis_index("c")
        iota = lax.iota(jnp.int32, _L)
        rows32 = [(iota + kk * _L) * D for kk in range(_BLK // _L)]

        def src_off(p):
            f = p // blocks_per_f
            blk = p % blocks_per_f
            return (f * B + blk * _BLK) * D, f, blk

        def fire(p, buf):
            off, _, _ = src_off(p)
            pltpu.async_copy(g_hbm.at[pl.ds(off, words)], buf, sem)

        def drain(p, buf):
            off, _, _ = src_off(p)
            pltpu.make_async_copy(g_hbm.at[pl.ds(off, words)], buf, sem).wait()

        def emit(p, buf):
            _, f, blk = src_off(p)
            for d in range(D):
                for kk in range(_BLK // _L):
                    vals = plsc.load_gather(buf, [rows32[kk] + d])
                    st[d, pl.ds(kk * _L, _L)] = vals
            pltpu.sync_copy(st, out_hbm.at[f, :, pl.ds(blk * _BLK, _BLK)])

        p0 = wid * n_pairs
        fire(p0, buf0)

        def step(i, carry):
            p = p0 + i * 2
            fire(p + 1, buf1)
            drain(p, buf0)
            emit(p, buf0)
            fire(p + 2, buf0)
            drain(p + 1, buf1)
            emit(p + 1, buf1)
            return carry

        lax.fori_loop(0, n_pairs // 2 - 1, step, 0)
        p = p0 + n_pairs - 2
        fire(p + 1, buf1)
        drain(p, buf0)
        emit(p, buf0)
        drain(p + 1, buf1)
        emit(p + 1, buf1)

    return pl.kernel(
        body,
        out_type=jax.ShapeDtypeStruct((F, D, B), jnp.float32),
        mesh=plsc.VectorSubcoreMesh(**_MESH),
        compiler_params=_TC_TILED,
        scratch_types=[
            pltpu.VMEM((words,), jnp.float32),
            pltpu.VMEM((words,), jnp.float32),
            pltpu.VMEM((D, _BLK), jnp.float32),
            pltpu.SemaphoreType.DMA,
        ],
    )


def kernel(x, weight):
    batch, n_fields = x.shape
    v, d = weight.shape
    n_full_rows = (v // _K0_CHUNK) * _K0_CHUNK
    w_tail = weight[n_full_rows:, :].reshape(-1)
    w_lin = _make_untile(v, d)(weight.T, w_tail).reshape(v, d)
    flat = x.T.reshape(-1)
    gath = _make_gather(flat.shape[0], v, d)(flat, w_lin)
    out = _make_retile(n_fields, batch, d)(gath.reshape(-1))
    return jnp.transpose(out, (2, 0, 1))


# speedup vs baseline: 1.2343x; 1.2343x over previous
import functools

import jax
import jax.numpy as jnp
from jax import lax
from jax.experimental import pallas as pl
from jax.experimental.pallas import tpu as pltpu
from jax.experimental.pallas import tpu_sc as plsc

_NC = 2
_NS = 16
_NW = _NC * _NS
_L = 16

_MESH = dict(core_axis_name="c", subcore_axis_name="s",
             num_cores=_NC, num_subcores=_NS)

_TC_TILED = pltpu.CompilerParams(use_tc_tiling_on_sc=True,
                                 needs_layout_passes=False)

_K0_CHUNK = 128


@functools.cache
def _make_untile(V, D):
    n_full = V // _K0_CHUNK
    n_pairs_total = n_full // 2
    pair_lo = n_pairs_total // _NW
    n_extra = n_pairs_total % _NW
    odd_chunk = n_full % 2
    tail = V % _K0_CHUNK
    words = _K0_CHUNK * D

    def body(wT_hbm, tail_hbm, out_hbm, buf0, buf1, st0, st1, sem):
        wid = lax.axis_index("s") * _NC + lax.axis_index("c")
        iota = lax.iota(jnp.int32, _L)

        extra = jnp.where(wid < n_extra, 1, 0)
        n_pairs = pair_lo + extra
        base = wid * pair_lo + jnp.minimum(wid, n_extra)

        def fire(c, buf):
            pltpu.async_copy(
                wT_hbm.at[:, pl.ds(c * _K0_CHUNK, _K0_CHUNK)], buf, sem)

        def drain(buf):
            pltpu.make_async_copy(
                wT_hbm.at[:, pl.ds(0, _K0_CHUNK)], buf, sem).wait()

        iota_d = iota * D

        def emit(c, buf, st):
            for d in range(D):
                for kk in range(_K0_CHUNK // _L):
                    vals = buf[d, pl.ds(kk * _L, _L)]
                    plsc.store_scatter(st, [iota_d + (kk * _L * D + d)], vals)
            pltpu.sync_copy(st, out_hbm.at[pl.ds(c * _K0_CHUNK * D, words)])

        fire(2 * base, buf0)

        def step(j, carry):
            c = 2 * (base + j)
            fire(c + 1, buf1)
            drain(buf0)
            emit(c, buf0, st0)
            nxt = 2 * (base + jnp.minimum(j + 1, n_pairs - 1))
            fire(nxt, buf0)
            drain(buf1)
            emit(c + 1, buf1, st1)
            return carry

        lax.fori_loop(0, n_pairs, step, 0)
        drain(buf0)

        if odd_chunk:
            @pl.when(wid == 1)
            def _():
                c = n_full - 1
                fire(c, buf0)
                drain(buf0)
                emit(c, buf0, st0)

        if tail:
            @pl.when(wid == 0)
            def _():
                pltpu.sync_copy(tail_hbm, st0.at[pl.ds(0, tail * D)])
                pltpu.sync_copy(st0.at[pl.ds(0, tail * D)],
                                out_hbm.at[pl.ds(n_full * _K0_CHUNK * D,
                                                 tail * D)])

    return pl.kernel(
        body,
        out_type=jax.ShapeDtypeStruct((V * D,), jnp.float32),
        mesh=plsc.VectorSubcoreMesh(**_MESH),
        compiler_params=_TC_TILED,
        scratch_types=[
            pltpu.VMEM((D, _K0_CHUNK), jnp.float32),
            pltpu.VMEM((D, _K0_CHUNK), jnp.float32),
            pltpu.VMEM((words,), jnp.float32),
            pltpu.VMEM((words,), jnp.float32),
            pltpu.SemaphoreType.DMA,
        ],
    )


_CHUNK = 1024
_GATHER = 128


@functools.cache
def _make_gather(N, V, D):
    per_w = N // _NW
    n_chunks = per_w // _CHUNK
    k = _CHUNK // _GATHER

    def body(idx_hbm, w_hbm, out_hbm, idx_v, rows_v, sem):
        wid = lax.axis_index("s") * _NC + lax.axis_index("c")
        base = wid * per_w

        def step(g, carry):
            off = base + g * _CHUNK
            pltpu.sync_copy(idx_hbm.at[pl.ds(off, _CHUNK)], idx_v)
            descs = [
                pltpu.async_copy(
                    w_hbm.at[idx_v.at[pl.ds(j * _GATHER, _GATHER)]],
                    rows_v.at[pl.ds(j * _GATHER, _GATHER)],
                    sem)
                for j in range(k)
            ]
            for d in descs:
                d.wait()
            pltpu.sync_copy(rows_v, out_hbm.at[pl.ds(off, _CHUNK)])
            return carry

        lax.fori_loop(0, n_chunks, step, 0)

    return pl.kernel(
        body,
        out_type=jax.ShapeDtypeStruct((N, D), jnp.float32),
        mesh=plsc.VectorSubcoreMesh(**_MESH),
        compiler_params=pltpu.CompilerParams(use_tc_tiling_on_sc=False),
        scratch_types=[
            pltpu.VMEM((_CHUNK,), jnp.int32),
            pltpu.VMEM((_CHUNK, D), jnp.float32),
            pltpu.SemaphoreType.DMA,
        ],
    )


_BLK = 128


@functools.cache
def _make_retile(F, B, D):
    n_pairs = F * (B // _BLK) // _NW
    blocks_per_f = B // _BLK
    words = _BLK * D

    def body(g_hbm, out_hbm, buf0, buf1, st, sem):
        wid = lax.axis_index("s") * _NC + lax.axis_index("c")
        iota = lax.iota(jnp.int32, _L)
        rows32 = [(iota + kk * _L) * D for kk in range(_BLK // _L)]

        def src_off(p):
            f = p // blocks_per_f
            blk = p % blocks_per_f
            return (f * B + blk * _BLK) * D, f, blk

        def fire(p, buf):
            off, _, _ = src_off(p)
            pltpu.async_copy(g_hbm.at[pl.ds(off, words)], buf, sem)

        def drain(p, buf):
            off, _, _ = src_off(p)
            pltpu.make_async_copy(g_hbm.at[pl.ds(off, words)], buf, sem).wait()

        def emit(p, buf):
            _, f, blk = src_off(p)
            for j in range(_BLK):
                for h in range(D // _L):
                    vals = buf[pl.ds(j * D + h * _L, _L)]
                    plsc.store_scatter(
                        st, [iota + h * _L, jnp.full((_L,), j, jnp.int32)],
                        vals)
            pltpu.sync_copy(st, out_hbm.at[f, :, pl.ds(blk * _BLK, _BLK)])

        p0 = wid * n_pairs
        fire(p0, buf0)

        def step(i, carry):
            p = p0 + i * 2
            fire(p + 1, buf1)
            drain(p, buf0)
            emit(p, buf0)
            fire(p + 2, buf0)
            drain(p + 1, buf1)
            emit(p + 1, buf1)
            return carry

        lax.fori_loop(0, n_pairs // 2 - 1, step, 0)
        p = p0 + n_pairs - 2
        fire(p + 1, buf1)
        drain(p, buf0)
        emit(p, buf0)
        drain(p + 1, buf1)
        emit(p + 1, buf1)

    return pl.kernel(
        body,
        out_type=jax.ShapeDtypeStruct((F, D, B), jnp.float32),
        mesh=plsc.VectorSubcoreMesh(**_MESH),
        compiler_params=_TC_TILED,
        scratch_types=[
            pltpu.VMEM((words,), jnp.float32),
            pltpu.VMEM((words,), jnp.float32),
            pltpu.VMEM((D, _BLK), jnp.float32),
            pltpu.SemaphoreType.DMA,
        ],
    )


def kernel(x, weight):
    batch, n_fields = x.shape
    v, d = weight.shape
    n_full_rows = (v // _K0_CHUNK) * _K0_CHUNK
    w_tail = weight[n_full_rows:, :].reshape(-1)
    w_lin = _make_untile(v, d)(weight.T, w_tail).reshape(v, d)
    flat = x.T.reshape(-1)
    gath = _make_gather(flat.shape[0], v, d)(flat, w_lin)
    out = _make_retile(n_fields, batch, d)(gath.reshape(-1))
    return jnp.transpose(out, (2, 0, 1))


# speedup vs baseline: 1.2423x; 1.0065x over previous
import functools

import jax
import jax.numpy as jnp
from jax import lax
from jax.experimental import pallas as pl
from jax.experimental.pallas import tpu as pltpu
from jax.experimental.pallas import tpu_sc as plsc

_NC = 2
_NS = 16
_NW = _NC * _NS
_L = 16

_MESH = dict(core_axis_name="c", subcore_axis_name="s",
             num_cores=_NC, num_subcores=_NS)

_TC_TILED = pltpu.CompilerParams(use_tc_tiling_on_sc=True,
                                 needs_layout_passes=False)

_K0_CHUNK = 512


@functools.cache
def _make_untile(V, D):
    n_full = V // _K0_CHUNK
    n_pairs_total = n_full // 2
    pair_lo = n_pairs_total // _NW
    n_extra = n_pairs_total % _NW
    odd_chunk = n_full % 2
    tail = V % _K0_CHUNK
    words = _K0_CHUNK * D

    def body(wT_hbm, tail_hbm, out_hbm, buf0, buf1, st0, st1, sem):
        wid = lax.axis_index("s") * _NC + lax.axis_index("c")
        iota = lax.iota(jnp.int32, _L)

        extra = jnp.where(wid < n_extra, 1, 0)
        n_pairs = pair_lo + extra
        base = wid * pair_lo + jnp.minimum(wid, n_extra)

        def fire(c, buf):
            pltpu.async_copy(
                wT_hbm.at[:, pl.ds(c * _K0_CHUNK, _K0_CHUNK)], buf, sem)

        def drain(buf):
            pltpu.make_async_copy(
                wT_hbm.at[:, pl.ds(0, _K0_CHUNK)], buf, sem).wait()

        iota_d = iota * D

        def emit(c, buf, st):
            def per_d(d, carry):
                for kk in range(_K0_CHUNK // _L):
                    vals = buf[d, pl.ds(kk * _L, _L)]
                    plsc.store_scatter(st, [iota_d + (kk * _L * D + d)], vals)
                return carry
            lax.fori_loop(0, D, per_d, 0)
            pltpu.sync_copy(st, out_hbm.at[pl.ds(c * _K0_CHUNK * D, words)])

        fire(2 * base, buf0)

        def step(j, carry):
            c = 2 * (base + j)
            fire(c + 1, buf1)
            drain(buf0)
            emit(c, buf0, st0)
            nxt = 2 * (base + jnp.minimum(j + 1, n_pairs - 1))
            fire(nxt, buf0)
            drain(buf1)
            emit(c + 1, buf1, st1)
            return carry

        lax.fori_loop(0, n_pairs, step, 0)
        drain(buf0)

        if odd_chunk:
            @pl.when(wid == 1)
            def _():
                c = n_full - 1
                fire(c, buf0)
                drain(buf0)
                emit(c, buf0, st0)

        if tail:
            @pl.when(wid == 0)
            def _():
                pltpu.sync_copy(tail_hbm, st0.at[pl.ds(0, tail * D)])
                pltpu.sync_copy(st0.at[pl.ds(0, tail * D)],
                                out_hbm.at[pl.ds(n_full * _K0_CHUNK * D,
                                                 tail * D)])

    return pl.kernel(
        body,
        out_type=jax.ShapeDtypeStruct((V * D,), jnp.float32),
        mesh=plsc.VectorSubcoreMesh(**_MESH),
        compiler_params=_TC_TILED,
        scratch_types=[
            pltpu.VMEM((D, _K0_CHUNK), jnp.float32),
            pltpu.VMEM((D, _K0_CHUNK), jnp.float32),
            pltpu.VMEM((words,), jnp.float32),
            pltpu.VMEM((words,), jnp.float32),
            pltpu.SemaphoreType.DMA,
        ],
    )


_CHUNK = 1024
_GATHER = 128


@functools.cache
def _make_gather(N, V, D):
    per_w = N // _NW
    n_chunks = per_w // _CHUNK
    k = _CHUNK // _GATHER

    def body(idx_hbm, w_hbm, out_hbm, idx_v, rows_v, sem):
        wid = lax.axis_index("s") * _NC + lax.axis_index("c")
        base = wid * per_w

        def step(g, carry):
            off = base + g * _CHUNK
            pltpu.sync_copy(idx_hbm.at[pl.ds(off, _CHUNK)], idx_v)
            descs = [
                pltpu.async_copy(
                    w_hbm.at[idx_v.at[pl.ds(j * _GATHER, _GATHER)]],
                    rows_v.at[pl.ds(j * _GATHER, _GATHER)],
                    sem)
                for j in range(k)
            ]
            for d in descs:
                d.wait()
            pltpu.sync_copy(rows_v, out_hbm.at[pl.ds(off, _CHUNK)])
            return carry

        lax.fori_loop(0, n_chunks, step, 0)

    return pl.kernel(
        body,
        out_type=jax.ShapeDtypeStruct((N, D), jnp.float32),
        mesh=plsc.VectorSubcoreMesh(**_MESH),
        compiler_params=pltpu.CompilerParams(use_tc_tiling_on_sc=False),
        scratch_types=[
            pltpu.VMEM((_CHUNK,), jnp.int32),
            pltpu.VMEM((_CHUNK, D), jnp.float32),
            pltpu.SemaphoreType.DMA,
        ],
    )


_BLK = 512


@functools.cache
def _make_retile(F, B, D):
    n_pairs = F * (B // _BLK) // _NW
    blocks_per_f = B // _BLK
    words = _BLK * D

    def body(g_hbm, out_hbm, buf0, buf1, st, sem):
        wid = lax.axis_index("s") * _NC + lax.axis_index("c")
        iota = lax.iota(jnp.int32, _L)

        def src_off(p):
            f = p // blocks_per_f
            blk = p % blocks_per_f
            return (f * B + blk * _BLK) * D, f, blk

        def fire(p, buf):
            off, _, _ = src_off(p)
            pltpu.async_copy(g_hbm.at[pl.ds(off, words)], buf, sem)

        def drain(p, buf):
            off, _, _ = src_off(p)
            pltpu.make_async_copy(g_hbm.at[pl.ds(off, words)], buf, sem).wait()

        def emit(p, buf):
            _, f, blk = src_off(p)

            def per_jb(jb, carry):
                for jj in range(_L):
                    j = jb * _L + jj
                    for h in range(D // _L):
                        vals = buf[pl.ds(j * D + h * _L, _L)]
                        plsc.store_scatter(
                            st,
                            [iota + h * _L, jnp.full((_L,), j, jnp.int32)],
                            vals)
                return carry
            lax.fori_loop(0, _BLK // _L, per_jb, 0)
            pltpu.sync_copy(st, out_hbm.at[f, :, pl.ds(blk * _BLK, _BLK)])

        p0 = wid * n_pairs
        fire(p0, buf0)

        def step(i, carry):
            p = p0 + i * 2
            fire(p + 1, buf1)
            drain(p, buf0)
            emit(p, buf0)
            fire(p + 2, buf0)
            drain(p + 1, buf1)
            emit(p + 1, buf1)
            return carry

        lax.fori_loop(0, n_pairs // 2 - 1, step, 0)
        p = p0 + n_pairs - 2
        fire(p + 1, buf1)
        drain(p, buf0)
        emit(p, buf0)
        drain(p + 1, buf1)
        emit(p + 1, buf1)

    return pl.kernel(
        body,
        out_type=jax.ShapeDtypeStruct((F, D, B), jnp.float32),
        mesh=plsc.VectorSubcoreMesh(**_MESH),
        compiler_params=_TC_TILED,
        scratch_types=[
            pltpu.VMEM((words,), jnp.float32),
            pltpu.VMEM((words,), jnp.float32),
            pltpu.VMEM((D, _BLK), jnp.float32),
            pltpu.SemaphoreType.DMA,
        ],
    )


def kernel(x, weight):
    batch, n_fields = x.shape
    v, d = weight.shape
    n_full_rows = (v // _K0_CHUNK) * _K0_CHUNK
    w_tail = weight[n_full_rows:, :].reshape(-1)
    w_lin = _make_untile(v, d)(weight.T, w_tail).reshape(v, d)
    flat = x.T.reshape(-1)
    gath = _make_gather(flat.shape[0], v, d)(flat, w_lin)
    out = _make_retile(n_fields, batch, d)(gath.reshape(-1))
    return jnp.transpose(out, (2, 0, 1))


# speedup vs baseline: 1.6451x; 1.3242x over previous
import functools

import jax
import jax.numpy as jnp
from jax import lax
from jax.experimental import pallas as pl
from jax.experimental.pallas import tpu as pltpu
from jax.experimental.pallas import tpu_sc as plsc

_NC = 2
_NS = 16
_NW = _NC * _NS
_L = 16

_MESH = dict(core_axis_name="c", subcore_axis_name="s",
             num_cores=_NC, num_subcores=_NS)

_TC_TILED = pltpu.CompilerParams(use_tc_tiling_on_sc=True,
                                 needs_layout_passes=False)

_K0_CHUNK = 512


@functools.cache
def _make_untile(V, D):
    n_full = V // _K0_CHUNK
    n_pairs_total = n_full // 2
    pair_lo = n_pairs_total // _NW
    n_extra = n_pairs_total % _NW
    odd_chunk = n_full % 2
    tail = V % _K0_CHUNK
    words = _K0_CHUNK * D

    def body(wT_hbm, tail_hbm, out_hbm, buf0, buf1, st0, st1, sem):
        wid = lax.axis_index("s") * _NC + lax.axis_index("c")
        iota = lax.iota(jnp.int32, _L)

        extra = jnp.where(wid < n_extra, 1, 0)
        n_pairs = pair_lo + extra
        base = wid * pair_lo + jnp.minimum(wid, n_extra)

        def fire(c, buf):
            pltpu.async_copy(
                wT_hbm.at[:, pl.ds(c * _K0_CHUNK, _K0_CHUNK)], buf, sem)

        def drain(buf):
            pltpu.make_async_copy(
                wT_hbm.at[:, pl.ds(0, _K0_CHUNK)], buf, sem).wait()

        iota_d = iota * D

        def emit(c, buf, st):
            @plsc.parallel_loop(0, D, 1, unroll=2)
            def per_d(d):
                for kk in range(_K0_CHUNK // _L):
                    vals = buf[d, pl.ds(kk * _L, _L)]
                    plsc.store_scatter(st, [iota_d + (kk * _L * D + d)], vals)
            pltpu.sync_copy(st, out_hbm.at[pl.ds(c * _K0_CHUNK * D, words)])

        fire(2 * base, buf0)

        def step(j, carry):
            c = 2 * (base + j)
            fire(c + 1, buf1)
            drain(buf0)
            emit(c, buf0, st0)
            nxt = 2 * (base + jnp.minimum(j + 1, n_pairs - 1))
            fire(nxt, buf0)
            drain(buf1)
            emit(c + 1, buf1, st1)
            return carry

        lax.fori_loop(0, n_pairs, step, 0)
        drain(buf0)

        if odd_chunk:
            @pl.when(wid == 1)
            def _():
                c = n_full - 1
                fire(c, buf0)
                drain(buf0)
                emit(c, buf0, st0)

        if tail:
            @pl.when(wid == 0)
            def _():
                pltpu.sync_copy(tail_hbm, st0.at[pl.ds(0, tail * D)])
                pltpu.sync_copy(st0.at[pl.ds(0, tail * D)],
                                out_hbm.at[pl.ds(n_full * _K0_CHUNK * D,
                                                 tail * D)])

    return pl.kernel(
        body,
        out_type=jax.ShapeDtypeStruct((V * D,), jnp.float32),
        mesh=plsc.VectorSubcoreMesh(**_MESH),
        compiler_params=_TC_TILED,
        scratch_types=[
            pltpu.VMEM((D, _K0_CHUNK), jnp.float32),
            pltpu.VMEM((D, _K0_CHUNK), jnp.float32),
            pltpu.VMEM((words,), jnp.float32),
            pltpu.VMEM((words,), jnp.float32),
            pltpu.SemaphoreType.DMA,
        ],
    )


_CHUNK = 1024
_GATHER = 128


@functools.cache
def _make_gather(N, V, D):
    per_w = N // _NW
    n_chunks = per_w // _CHUNK
    k = _CHUNK // _GATHER

    def body(idx_hbm, w_hbm, out_hbm, idx_v, rows_v, sem):
        wid = lax.axis_index("s") * _NC + lax.axis_index("c")
        base = wid * per_w

        def step(g, carry):
            off = base + g * _CHUNK
            pltpu.sync_copy(idx_hbm.at[pl.ds(off, _CHUNK)], idx_v)
            descs = [
                pltpu.async_copy(
                    w_hbm.at[idx_v.at[pl.ds(j * _GATHER, _GATHER)]],
                    rows_v.at[pl.ds(j * _GATHER, _GATHER)],
                    sem)
                for j in range(k)
            ]
            for d in descs:
                d.wait()
            pltpu.sync_copy(rows_v, out_hbm.at[pl.ds(off, _CHUNK)])
            return carry

        lax.fori_loop(0, n_chunks, step, 0)

    return pl.kernel(
        body,
        out_type=jax.ShapeDtypeStruct((N, D), jnp.float32),
        mesh=plsc.VectorSubcoreMesh(**_MESH),
        compiler_params=pltpu.CompilerParams(use_tc_tiling_on_sc=False),
        scratch_types=[
            pltpu.VMEM((_CHUNK,), jnp.int32),
            pltpu.VMEM((_CHUNK, D), jnp.float32),
            pltpu.SemaphoreType.DMA,
        ],
    )


_BLK = 512


@functools.cache
def _make_retile(F, B, D):
    n_pairs = F * (B // _BLK) // _NW
    blocks_per_f = B // _BLK
    words = _BLK * D

    def body(g_hbm, out_hbm, buf0, buf1, st, sem):
        wid = lax.axis_index("s") * _NC + lax.axis_index("c")
        iota = lax.iota(jnp.int32, _L)

        def src_off(p):
            f = p // blocks_per_f
            blk = p % blocks_per_f
            return (f * B + blk * _BLK) * D, f, blk

        def fire(p, buf):
            off, _, _ = src_off(p)
            pltpu.async_copy(g_hbm.at[pl.ds(off, words)], buf, sem)

        def drain(p, buf):
            off, _, _ = src_off(p)
            pltpu.make_async_copy(g_hbm.at[pl.ds(off, words)], buf, sem).wait()

        def emit(p, buf):
            _, f, blk = src_off(p)

            @plsc.parallel_loop(0, _BLK // _L, 1, unroll=2)
            def per_jb(jb):
                for jj in range(_L):
                    j = jb * _L + jj
                    for h in range(D // _L):
                        vals = buf[pl.ds(j * D + h * _L, _L)]
                        plsc.store_scatter(
                            st,
                            [iota + h * _L, jnp.full((_L,), j, jnp.int32)],
                            vals)
            pltpu.sync_copy(st, out_hbm.at[f, :, pl.ds(blk * _BLK, _BLK)])

        p0 = wid * n_pairs
        fire(p0, buf0)

        def step(i, carry):
            p = p0 + i * 2
            fire(p + 1, buf1)
            drain(p, buf0)
            emit(p, buf0)
            fire(p + 2, buf0)
            drain(p + 1, buf1)
            emit(p + 1, buf1)
            return carry

        lax.fori_loop(0, n_pairs // 2 - 1, step, 0)
        p = p0 + n_pairs - 2
        fire(p + 1, buf1)
        drain(p, buf0)
        emit(p, buf0)
        drain(p + 1, buf1)
        emit(p + 1, buf1)

    return pl.kernel(
        body,
        out_type=jax.ShapeDtypeStruct((F, D, B), jnp.float32),
        mesh=plsc.VectorSubcoreMesh(**_MESH),
        compiler_params=_TC_TILED,
        scratch_types=[
            pltpu.VMEM((words,), jnp.float32),
            pltpu.VMEM((words,), jnp.float32),
            pltpu.VMEM((D, _BLK), jnp.float32),
            pltpu.SemaphoreType.DMA,
        ],
    )


def kernel(x, weight):
    batch, n_fields = x.shape
    v, d = weight.shape
    n_full_rows = (v // _K0_CHUNK) * _K0_CHUNK
    w_tail = weight[n_full_rows:, :].reshape(-1)
    w_lin = _make_untile(v, d)(weight.T, w_tail).reshape(v, d)
    flat = x.T.reshape(-1)
    gath = _make_gather(flat.shape[0], v, d)(flat, w_lin)
    out = _make_retile(n_fields, batch, d)(gath.reshape(-1))
    return jnp.transpose(out, (2, 0, 1))


# speedup vs baseline: 1.7048x; 1.0363x over previous
import functools

import jax
import jax.numpy as jnp
from jax import lax
from jax.experimental import pallas as pl
from jax.experimental.pallas import tpu as pltpu
from jax.experimental.pallas import tpu_sc as plsc

_NC = 2
_NS = 16
_NW = _NC * _NS
_L = 16

_MESH = dict(core_axis_name="c", subcore_axis_name="s",
             num_cores=_NC, num_subcores=_NS)

_TC_TILED = pltpu.CompilerParams(use_tc_tiling_on_sc=True,
                                 needs_layout_passes=False)

_K0_CHUNK = 512


@functools.cache
def _make_untile(V, D):
    n_full = V // _K0_CHUNK
    n_pairs_total = n_full // 2
    pair_lo = n_pairs_total // _NW
    n_extra = n_pairs_total % _NW
    odd_chunk = n_full % 2
    tail = V % _K0_CHUNK
    words = _K0_CHUNK * D

    def body(wT_hbm, tail_hbm, out_hbm, buf0, buf1, st0, st1, sem, wsem):
        wid = lax.axis_index("s") * _NC + lax.axis_index("c")
        iota = lax.iota(jnp.int32, _L)

        def wdrain():
            pltpu.make_async_copy(
                st0, out_hbm.at[pl.ds(0, words)], wsem).wait()

        extra = jnp.where(wid < n_extra, 1, 0)
        n_pairs = pair_lo + extra
        base = wid * pair_lo + jnp.minimum(wid, n_extra)

        def fire(c, buf):
            pltpu.async_copy(
                wT_hbm.at[:, pl.ds(c * _K0_CHUNK, _K0_CHUNK)], buf, sem)

        def drain(buf):
            pltpu.make_async_copy(
                wT_hbm.at[:, pl.ds(0, _K0_CHUNK)], buf, sem).wait()

        iota_d = iota * D

        def emit(c, buf, st):
            @plsc.parallel_loop(0, D, 1, unroll=2)
            def per_d(d):
                for kk in range(_K0_CHUNK // _L):
                    vals = buf[d, pl.ds(kk * _L, _L)]
                    plsc.store_scatter(st, [iota_d + (kk * _L * D + d)], vals)
            pltpu.async_copy(st, out_hbm.at[pl.ds(c * _K0_CHUNK * D, words)],
                             wsem)

        fire(2 * base, buf0)

        def step(j, carry):
            c = 2 * (base + j)
            fire(c + 1, buf1)
            drain(buf0)

            @pl.when(j > 0)
            def _():
                wdrain()
                wdrain()
            emit(c, buf0, st0)
            nxt = 2 * (base + jnp.minimum(j + 1, n_pairs - 1))
            fire(nxt, buf0)
            drain(buf1)
            emit(c + 1, buf1, st1)
            return carry

        lax.fori_loop(0, n_pairs, step, 0)
        drain(buf0)
        wdrain()
        wdrain()

        if odd_chunk:
            @pl.when(wid == 1)
            def _():
                c = n_full - 1
                fire(c, buf0)
                drain(buf0)
                emit(c, buf0, st0)
                wdrain()

        if tail:
            @pl.when(wid == 0)
            def _():
                pltpu.sync_copy(tail_hbm, st0.at[pl.ds(0, tail * D)])
                pltpu.sync_copy(st0.at[pl.ds(0, tail * D)],
                                out_hbm.at[pl.ds(n_full * _K0_CHUNK * D,
                                                 tail * D)])

    return pl.kernel(
        body,
        out_type=jax.ShapeDtypeStruct((V * D,), jnp.float32),
        mesh=plsc.VectorSubcoreMesh(**_MESH),
        compiler_params=_TC_TILED,
        scratch_types=[
            pltpu.VMEM((D, _K0_CHUNK), jnp.float32),
            pltpu.VMEM((D, _K0_CHUNK), jnp.float32),
            pltpu.VMEM((words,), jnp.float32),
            pltpu.VMEM((words,), jnp.float32),
            pltpu.SemaphoreType.DMA,
            pltpu.SemaphoreType.DMA,
        ],
    )


_CHUNK = 1024
_GATHER = 128


@functools.cache
def _make_gather(N, V, D):
    per_w = N // _NW
    n_chunks = per_w // _CHUNK
    k = _CHUNK // _GATHER

    def body(idx_hbm, w_hbm, out_hbm, idx_v, rows_v, sem):
        wid = lax.axis_index("s") * _NC + lax.axis_index("c")
        base = wid * per_w

        def step(g, carry):
            off = base + g * _CHUNK
            pltpu.sync_copy(idx_hbm.at[pl.ds(off, _CHUNK)], idx_v)
            descs = [
                pltpu.async_copy(
                    w_hbm.at[idx_v.at[pl.ds(j * _GATHER, _GATHER)]],
                    rows_v.at[pl.ds(j * _GATHER, _GATHER)],
                    sem)
                for j in range(k)
            ]
            for d in descs:
                d.wait()
            pltpu.sync_copy(rows_v, out_hbm.at[pl.ds(off, _CHUNK)])
            return carry

        lax.fori_loop(0, n_chunks, step, 0)

    return pl.kernel(
        body,
        out_type=jax.ShapeDtypeStruct((N, D), jnp.float32),
        mesh=plsc.VectorSubcoreMesh(**_MESH),
        compiler_params=pltpu.CompilerParams(use_tc_tiling_on_sc=False),
        scratch_types=[
            pltpu.VMEM((_CHUNK,), jnp.int32),
            pltpu.VMEM((_CHUNK, D), jnp.float32),
            pltpu.SemaphoreType.DMA,
        ],
    )


_BLK = 512


@functools.cache
def _make_retile(F, B, D):
    n_pairs = F * (B // _BLK) // _NW
    blocks_per_f = B // _BLK
    words = _BLK * D

    def body(g_hbm, out_hbm, buf0, buf1, st0, st1, sem, wsem):
        wid = lax.axis_index("s") * _NC + lax.axis_index("c")
        iota = lax.iota(jnp.int32, _L)

        def wdrain():
            pltpu.make_async_copy(
                st0, out_hbm.at[0, :, pl.ds(0, _BLK)], wsem).wait()

        def src_off(p):
            f = p // blocks_per_f
            blk = p % blocks_per_f
            return (f * B + blk * _BLK) * D, f, blk

        def fire(p, buf):
            off, _, _ = src_off(p)
            pltpu.async_copy(g_hbm.at[pl.ds(off, words)], buf, sem)

        def drain(p, buf):
            off, _, _ = src_off(p)
            pltpu.make_async_copy(g_hbm.at[pl.ds(off, words)], buf, sem).wait()

        def emit(p, buf, st):
            _, f, blk = src_off(p)

            @plsc.parallel_loop(0, _BLK // _L, 1, unroll=2)
            def per_jb(jb):
                for jj in range(_L):
                    j = jb * _L + jj
                    for h in range(D // _L):
                        vals = buf[pl.ds(j * D + h * _L, _L)]
                        plsc.store_scatter(
                            st,
                            [iota + h * _L, jnp.full((_L,), j, jnp.int32)],
                            vals)
            pltpu.async_copy(st, out_hbm.at[f, :, pl.ds(blk * _BLK, _BLK)],
                             wsem)

        p0 = wid * n_pairs
        fire(p0, buf0)

        def step(i, carry):
            p = p0 + i * 2
            fire(p + 1, buf1)
            drain(p, buf0)

            @pl.when(i > 0)
            def _():
                wdrain()
                wdrain()
            emit(p, buf0, st0)
            fire(p + 2, buf0)
            drain(p + 1, buf1)
            emit(p + 1, buf1, st1)
            return carry

        lax.fori_loop(0, n_pairs // 2 - 1, step, 0)
        p = p0 + n_pairs - 2
        fire(p + 1, buf1)
        drain(p, buf0)
        wdrain()
        wdrain()
        emit(p, buf0, st0)
        drain(p + 1, buf1)
        emit(p + 1, buf1, st1)
        wdrain()
        wdrain()

    return pl.kernel(
        body,
        out_type=jax.ShapeDtypeStruct((F, D, B), jnp.float32),
        mesh=plsc.VectorSubcoreMesh(**_MESH),
        compiler_params=_TC_TILED,
        scratch_types=[
            pltpu.VMEM((words,), jnp.float32),
            pltpu.VMEM((words,), jnp.float32),
            pltpu.VMEM((D, _BLK), jnp.float32),
            pltpu.VMEM((D, _BLK), jnp.float32),
            pltpu.SemaphoreType.DMA,
            pltpu.SemaphoreType.DMA,
        ],
    )


def kernel(x, weight):
    batch, n_fields = x.shape
    v, d = weight.shape
    n_full_rows = (v // _K0_CHUNK) * _K0_CHUNK
    w_tail = weight[n_full_rows:, :].reshape(-1)
    w_lin = _make_untile(v, d)(weight.T, w_tail).reshape(v, d)
    flat = x.T.reshape(-1)
    gath = _make_gather(flat.shape[0], v, d)(flat, w_lin)
    out = _make_retile(n_fields, batch, d)(gath.reshape(-1))
    return jnp.transpose(out, (2, 0, 1))
